# baseline (speedup 1.0000x reference)
import functools

import jax
import jax.numpy as jnp
import numpy as np
from jax import lax
from jax.experimental import pallas as pl
from jax.experimental.pallas import tpu as pltpu
from jax.experimental.pallas import tpu_sc as plsc

N = 10000
C = 128
K = 32
G = 4
NPAD = 10240
NSUB = 16
CHUNK_NODES = 2
CHN = CHUNK_NODES * K
TOT_CHUNKS = NPAD // CHUNK_NODES
NLANE = 16
NV = C // NLANE
NBUF = 4

NWORK = 2 * NSUB
CPW = TOT_CHUNKS // NWORK


TRB = 1024


def _transpose_body(x_ref, o_ref):
    o_ref[...] = x_ref[...].T


def _transpose_cn_to_nc(x_sq):
    return pl.pallas_call(
        _transpose_body,
        grid=(NPAD // TRB,),
        in_specs=[pl.BlockSpec((C, TRB), lambda i: (0, i))],
        out_specs=pl.BlockSpec((TRB, C), lambda i: (i, 0)),
        out_shape=jax.ShapeDtypeStruct((NPAD, C), jnp.float32),
    )(x_sq)


def _sc_gather_sum(xT_hbm, idx_hbm, out_hbm, idx_v,
                   buf0, buf1, buf2, buf3,
                   out_v, sem0, sem1, sem2, sem3):
    bufs = (buf0, buf1, buf2, buf3)
    sems = (sem0, sem1, sem2, sem3)
    cid = lax.axis_index("c")
    sid = lax.axis_index("s")

    def gather(c, buf, sem):
        pltpu.make_async_copy(xT_hbm.at[idx_v.at[c]], buf, sem).start()

    def wait(buf, sem):
        pltpu.make_async_copy(xT_hbm.at[idx_v.at[0]], buf, sem).wait()

    def compute_chunk(buf, c):
        for i in range(CHUNK_NODES):
            accs = [buf[i * K, pl.ds(j * NLANE, NLANE)] for j in range(NV)]
            for rr in range(1, K):
                row = i * K + rr
                for j in range(NV):
                    accs[j] = accs[j] + buf[row, pl.ds(j * NLANE, NLANE)]
            node = c * CHUNK_NODES + i
            for j in range(NV):
                out_v[node, pl.ds(j * NLANE, NLANE)] = accs[j]

    wid = cid * NSUB + sid
    chunk0 = wid * CPW

    pltpu.sync_copy(idx_hbm.at[pl.ds(chunk0, CPW)], idx_v)

    for b in range(NBUF):
        gather(b, bufs[b], sems[b])

    def step(p, carry):
        for b in range(NBUF):
            c = NBUF * p + b
            wait(bufs[b], sems[b])
            compute_chunk(bufs[b], c)

            @pl.when(p < CPW // NBUF - 1)
            def _(b=b, c=c):
                gather(c + NBUF, bufs[b], sems[b])
        return carry

    lax.fori_loop(0, CPW // NBUF, step, 0)

    pltpu.sync_copy(out_v, out_hbm.at[pl.ds(chunk0 * CHUNK_NODES,
                                            CPW * CHUNK_NODES)])


def _neighbor_sum(xT, idx2):
    mesh = plsc.VectorSubcoreMesh(core_axis_name="c", subcore_axis_name="s",
                                  num_cores=2, num_subcores=NSUB)
    kern = functools.partial(
        pl.kernel,
        out_type=jax.ShapeDtypeStruct((NPAD, C), jnp.float32),
        mesh=mesh,
        compiler_params=pltpu.CompilerParams(use_tc_tiling_on_sc=True),
        scratch_types=(
            [pltpu.VMEM((CPW, CHN), jnp.int32)]
            + [pltpu.VMEM((CHN, C), jnp.float32) for _ in range(NBUF)]
            + [pltpu.VMEM((CPW * CHUNK_NODES, C), jnp.float32)]
            + [pltpu.SemaphoreType.DMA for _ in range(NBUF)]
        ),
    )(_sc_gather_sum)
    return kern(xT, idx2)


def _conv_body(eps_ref, x_ref, xj_ref, W_ref, b_ref, o_ref):
    scale = 1.0 + eps_ref[0]
    h = scale * x_ref[...].T + xj_ref[...]
    y = lax.dot_general(W_ref[...], h, (((1,), (1,)), ((), ())),
                        preferred_element_type=jnp.float32)
    o_ref[...] = jnp.maximum(y + b_ref[...], 0.0)


def _gin_update(eps, x_sq, xj, W_bd, b):
    return pl.pallas_call(
        _conv_body,
        grid=(NPAD // TRB,),
        in_specs=[
            pl.BlockSpec(memory_space=pltpu.SMEM),
            pl.BlockSpec((C, TRB), lambda i: (0, i)),
            pl.BlockSpec((TRB, C), lambda i: (i, 0)),
            pl.BlockSpec((C, C), lambda i: (0, 0)),
            pl.BlockSpec((C, 1), lambda i: (0, 0)),
        ],
        out_specs=pl.BlockSpec((C, TRB), lambda i: (0, i)),
        out_shape=jax.ShapeDtypeStruct((C, N), jnp.float32),
    )(eps, x_sq, xj, W_bd, b)


def kernel(x, edge_index, W, b, eps):
    x_sq = x[0, :, :, 0]
    idx = edge_index[0, 0]
    fill = (jnp.arange(NPAD - N, dtype=jnp.int32)[:, None] * K
            + jnp.arange(K, dtype=jnp.int32)[None, :]) % N
    idx_pad = jnp.concatenate([idx, fill], axis=0)
    idx2 = idx_pad.reshape(TOT_CHUNKS, CHN)

    Wg = W[:, :, 0, 0]
    W_bd = jnp.zeros((C, C), jnp.float32)
    gs = C // G
    for g in range(G):
        W_bd = W_bd.at[g * gs:(g + 1) * gs, g * gs:(g + 1) * gs].set(
            Wg[g * gs:(g + 1) * gs, :])

    xT = _transpose_cn_to_nc(x_sq)
    xj = _neighbor_sum(xT, idx2)
    out = _gin_update(eps, x_sq, xj, W_bd, b[:, None])
    return out[None, :, :, None]


# speedup vs baseline: 1.6756x; 1.6756x over previous
import functools

import jax
import jax.numpy as jnp
import numpy as np
from jax import lax
from jax.experimental import pallas as pl
from jax.experimental.pallas import tpu as pltpu
from jax.experimental.pallas import tpu_sc as plsc

N = 10000
C = 128
K = 32
G = 4
NPAD = 10240
NSUB = 16
CHUNK_NODES = 2
CHN = CHUNK_NODES * K
TOT_CHUNKS = NPAD // CHUNK_NODES
NLANE = 16
NV = C // NLANE
NBUF = 4

NWORK = 2 * NSUB
CPW = TOT_CHUNKS // NWORK


TRB = 1024


def _transpose_body(x_ref, o_ref):
    o_ref[...] = x_ref[...].T


def _transpose_cn_to_nc(x_sq):
    return pl.pallas_call(
        _transpose_body,
        grid=(NPAD // TRB,),
        in_specs=[pl.BlockSpec((C, TRB), lambda i: (0, i))],
        out_specs=pl.BlockSpec((TRB, C), lambda i: (i, 0)),
        out_shape=jax.ShapeDtypeStruct((NPAD, C), jnp.float32),
    )(x_sq)


def _sc_gather_sum(xT_hbm, idx_hbm, out_hbm, idx_v,
                   buf0, buf1, buf2, buf3,
                   out_v, sem0, sem1, sem2, sem3):
    bufs = (buf0, buf1, buf2, buf3)
    sems = (sem0, sem1, sem2, sem3)
    cid = lax.axis_index("c")
    sid = lax.axis_index("s")

    def gather(c, buf, sem):
        pltpu.make_async_copy(xT_hbm.at[idx_v.at[c]], buf, sem).start()

    def wait(buf, sem):
        pltpu.make_async_copy(xT_hbm.at[idx_v.at[0]], buf, sem).wait()

    def compute_chunk(buf, c):
        for i in range(CHUNK_NODES):
            def rbody(q, accs, _i=i):
                base = _i * K + q * 8
                for rr in range(8):
                    row = base + rr
                    accs = tuple(
                        accs[j] + buf[row, pl.ds(j * NLANE, NLANE)]
                        for j in range(NV)
                    )
                return accs
            accs0 = tuple(jnp.zeros((NLANE,), jnp.float32) for _ in range(NV))
            accs = lax.fori_loop(0, K // 8, rbody, accs0)
            node = c * CHUNK_NODES + i
            for j in range(NV):
                out_v[node, pl.ds(j * NLANE, NLANE)] = accs[j]

    wid = cid * NSUB + sid
    chunk0 = wid * CPW

    pltpu.sync_copy(idx_hbm.at[pl.ds(chunk0, CPW)], idx_v)

    for b in range(NBUF):
        gather(b, bufs[b], sems[b])

    def step(p, carry):
        for b in range(NBUF):
            c = NBUF * p + b
            wait(bufs[b], sems[b])
            compute_chunk(bufs[b], c)

            @pl.when(p < CPW // NBUF - 1)
            def _(b=b, c=c):
                gather(c + NBUF, bufs[b], sems[b])
        return carry

    lax.fori_loop(0, CPW // NBUF, step, 0)

    pltpu.sync_copy(out_v, out_hbm.at[pl.ds(chunk0 * CHUNK_NODES,
                                            CPW * CHUNK_NODES)])


def _neighbor_sum(xT, idx2):
    mesh = plsc.VectorSubcoreMesh(core_axis_name="c", subcore_axis_name="s",
                                  num_cores=2, num_subcores=NSUB)
    kern = functools.partial(
        pl.kernel,
        out_type=jax.ShapeDtypeStruct((NPAD, C), jnp.float32),
        mesh=mesh,
        scratch_types=(
            [pltpu.VMEM((CPW, CHN), jnp.int32)]
            + [pltpu.VMEM((CHN, C), jnp.float32) for _ in range(NBUF)]
            + [pltpu.VMEM((CPW * CHUNK_NODES, C), jnp.float32)]
            + [pltpu.SemaphoreType.DMA for _ in range(NBUF)]
        ),
    )(_sc_gather_sum)
    return kern(xT, idx2)


def _conv_body(eps_ref, x_ref, xj_ref, W_ref, b_ref, o_ref):
    scale = 1.0 + eps_ref[0]
    h = scale * x_ref[...].T + xj_ref[...]
    y = lax.dot_general(W_ref[...], h, (((1,), (1,)), ((), ())),
                        preferred_element_type=jnp.float32)
    o_ref[...] = jnp.maximum(y + b_ref[...], 0.0)


def _gin_update(eps, x_sq, xj, W_bd, b):
    return pl.pallas_call(
        _conv_body,
        grid=(NPAD // TRB,),
        in_specs=[
            pl.BlockSpec(memory_space=pltpu.SMEM),
            pl.BlockSpec((C, TRB), lambda i: (0, i)),
            pl.BlockSpec((TRB, C), lambda i: (i, 0)),
            pl.BlockSpec((C, C), lambda i: (0, 0)),
            pl.BlockSpec((C, 1), lambda i: (0, 0)),
        ],
        out_specs=pl.BlockSpec((C, TRB), lambda i: (0, i)),
        out_shape=jax.ShapeDtypeStruct((C, N), jnp.float32),
    )(eps, x_sq, xj, W_bd, b)


def kernel(x, edge_index, W, b, eps):
    x_sq = x[0, :, :, 0]
    idx = edge_index[0, 0]
    fill = (jnp.arange(NPAD - N, dtype=jnp.int32)[:, None] * K
            + jnp.arange(K, dtype=jnp.int32)[None, :]) % N
    idx_pad = jnp.concatenate([idx, fill], axis=0)
    idx2 = idx_pad.reshape(TOT_CHUNKS, CHN)

    Wg = W[:, :, 0, 0]
    W_bd = jnp.zeros((C, C), jnp.float32)
    gs = C // G
    for g in range(G):
        W_bd = W_bd.at[g * gs:(g + 1) * gs, g * gs:(g + 1) * gs].set(
            Wg[g * gs:(g + 1) * gs, :])

    xT = _transpose_cn_to_nc(x_sq)
    xj = _neighbor_sum(xT, idx2)
    out = _gin_update(eps, x_sq, xj, W_bd, b[:, None])
    return out[None, :, :, None]


# speedup vs baseline: 2.1796x; 1.3008x over previous
import functools

import jax
import jax.numpy as jnp
import numpy as np
from jax import lax
from jax.experimental import pallas as pl
from jax.experimental.pallas import tpu as pltpu
from jax.experimental.pallas import tpu_sc as plsc

N = 10000
C = 128
K = 32
G = 4
NPAD = 10240
NSUB = 16
CHUNK_NODES = 2
CHN = CHUNK_NODES * K
TOT_CHUNKS = NPAD // CHUNK_NODES
NLANE = 16
NV = C // NLANE
NBUF = 4

NWORK = 2 * NSUB
CPW = TOT_CHUNKS // NWORK


TRB = 1024


def _sc_gather_sum(xT_hbm, idx_hbm, out_hbm, idx_v,
                   buf0, buf1, buf2, buf3,
                   out_v, sem0, sem1, sem2, sem3):
    bufs = (buf0, buf1, buf2, buf3)
    sems = (sem0, sem1, sem2, sem3)
    cid = lax.axis_index("c")
    sid = lax.axis_index("s")

    def gather(c, buf, sem):
        pltpu.make_async_copy(xT_hbm.at[idx_v.at[c]], buf, sem).start()

    def wait(buf, sem):
        pltpu.make_async_copy(xT_hbm.at[idx_v.at[0]], buf, sem).wait()

    def compute_chunk(buf, c):
        for i in range(CHUNK_NODES):
            def rbody(q, accs, _i=i):
                base = _i * K + q * 8
                for rr in range(8):
                    row = base + rr
                    accs = tuple(
                        accs[j] + buf[row, pl.ds(j * NLANE, NLANE)]
                        for j in range(NV)
                    )
                return accs
            accs0 = tuple(jnp.zeros((NLANE,), jnp.float32) for _ in range(NV))
            accs = lax.fori_loop(0, K // 8, rbody, accs0)
            node = c * CHUNK_NODES + i
            for j in range(NV):
                out_v[node, pl.ds(j * NLANE, NLANE)] = accs[j]

    wid = cid * NSUB + sid
    chunk0 = wid * CPW

    pltpu.sync_copy(idx_hbm.at[pl.ds(chunk0, CPW)], idx_v)

    for b in range(NBUF):
        gather(b, bufs[b], sems[b])

    def step(p, carry):
        for b in range(NBUF):
            c = NBUF * p + b
            wait(bufs[b], sems[b])
            compute_chunk(bufs[b], c)

            @pl.when(p < CPW // NBUF - 1)
            def _(b=b, c=c):
                gather(c + NBUF, bufs[b], sems[b])
        return carry

    lax.fori_loop(0, CPW // NBUF, step, 0)

    pltpu.sync_copy(out_v, out_hbm.at[pl.ds(chunk0 * CHUNK_NODES,
                                            CPW * CHUNK_NODES)])


def _neighbor_sum(xT, idx2):
    mesh = plsc.VectorSubcoreMesh(core_axis_name="c", subcore_axis_name="s",
                                  num_cores=2, num_subcores=NSUB)
    kern = functools.partial(
        pl.kernel,
        out_type=jax.ShapeDtypeStruct((NPAD, C), jnp.float32),
        mesh=mesh,
        scratch_types=(
            [pltpu.VMEM((CPW, CHN), jnp.int32)]
            + [pltpu.VMEM((CHN, C), jnp.float32) for _ in range(NBUF)]
            + [pltpu.VMEM((CPW * CHUNK_NODES, C), jnp.float32)]
            + [pltpu.SemaphoreType.DMA for _ in range(NBUF)]
        ),
    )(_sc_gather_sum)
    return kern(xT, idx2)


def _conv_body(eps_ref, xT_ref, xj_ref, W_ref, b_ref, o_ref):
    scale = 1.0 + eps_ref[0]
    h = scale * xT_ref[...] + xj_ref[...]
    y = lax.dot_general(h, W_ref[...], (((1,), (1,)), ((), ())),
                        preferred_element_type=jnp.float32)
    o_ref[...] = jnp.maximum(y + b_ref[...], 0.0)


def _gin_update(eps, xT, xj, W_bd, b):
    return pl.pallas_call(
        _conv_body,
        grid=(NPAD // TRB,),
        in_specs=[
            pl.BlockSpec(memory_space=pltpu.SMEM),
            pl.BlockSpec((TRB, C), lambda i: (i, 0)),
            pl.BlockSpec((TRB, C), lambda i: (i, 0)),
            pl.BlockSpec((C, C), lambda i: (0, 0)),
            pl.BlockSpec((1, C), lambda i: (0, 0)),
        ],
        out_specs=pl.BlockSpec((TRB, C), lambda i: (i, 0)),
        out_shape=jax.ShapeDtypeStruct((N, C), jnp.float32),
    )(eps, xT, xj, W_bd, b)


def kernel(x, edge_index, W, b, eps):
    xT = x[0, :, :, 0].T
    idx = edge_index[0, 0]
    fill = (jnp.arange(NPAD - N, dtype=jnp.int32)[:, None] * K
            + jnp.arange(K, dtype=jnp.int32)[None, :]) % N
    idx_pad = jnp.concatenate([idx, fill], axis=0)
    idx2 = idx_pad.reshape(TOT_CHUNKS, CHN)

    Wg = W[:, :, 0, 0]
    W_bd = jnp.zeros((C, C), jnp.float32)
    gs = C // G
    for g in range(G):
        W_bd = W_bd.at[g * gs:(g + 1) * gs, g * gs:(g + 1) * gs].set(
            Wg[g * gs:(g + 1) * gs, :])

    xj = _neighbor_sum(xT, idx2)
    out = _gin_update(eps, xT, xj, W_bd, b[None, :])
    return out.T[None, :, :, None]
